# Initial kernel scaffold; baseline (speedup 1.0000x reference)
import jax
import jax.numpy as jnp
from jax.experimental import pallas as pl


def kernel(fg_proto, bg_proto, F_map, M_map):
    raise NotImplementedError("write your pallas kernel here")


# speedup vs baseline: 8.5109x; 8.5109x over previous
import functools

import jax
import jax.numpy as jnp
from jax.experimental import pallas as pl

BETA = 1.0
M_COEF = 0.3
LR_REFINE = 0.1
N_REFINE = 10
N_SAMPLES = 1024
MARGIN = 0.2
TAU = 0.07
K_PROTO = 512
D_FEAT = 256

_ROWS_BLK = 2048


def _score_body(ft_ref, p_ref, m_ref, fn_ref, sc_ref):
    x = ft_ref[...]
    nrm = jnp.sqrt(jnp.sum(x * x, axis=1, keepdims=True))
    f = x / jnp.maximum(nrm, 1e-8)
    fn_ref[...] = f
    d = jax.lax.dot_general(
        f, p_ref[...], (((1,), (1,)), ((), ())),
        preferred_element_type=jnp.float32)
    mxf = jnp.max(d[:, :K_PROTO], axis=1, keepdims=True)
    mxb = jnp.max(d[:, K_PROTO:], axis=1, keepdims=True)
    m = m_ref[...]
    sc_ref[...] = jnp.concatenate(
        [(1.0 - mxf) * m, (1.0 - mxb) * (1.0 - m)], axis=1)


def _scores_and_norm(ft, protos, m_col):
    n_rows = ft.shape[0]
    grid = (n_rows // _ROWS_BLK,)
    return pl.pallas_call(
        _score_body,
        grid=grid,
        in_specs=[
            pl.BlockSpec((_ROWS_BLK, D_FEAT), lambda i: (i, 0)),
            pl.BlockSpec((2 * K_PROTO, D_FEAT), lambda i: (0, 0)),
            pl.BlockSpec((_ROWS_BLK, 1), lambda i: (i, 0)),
        ],
        out_specs=[
            pl.BlockSpec((_ROWS_BLK, D_FEAT), lambda i: (i, 0)),
            pl.BlockSpec((_ROWS_BLK, 2), lambda i: (i, 0)),
        ],
        out_shape=[
            jax.ShapeDtypeStruct((n_rows, D_FEAT), jnp.float32),
            jax.ShapeDtypeStruct((n_rows, 2), jnp.float32),
        ],
    )(ft, protos, m_col)


def _safe_norm(x):
    n = jnp.sqrt(jnp.sum(x * x, axis=-1, keepdims=True))
    return x / jnp.maximum(n, 1e-8)


def _refine_unrolled(p, feats):
    n = feats.shape[0]
    ones = jnp.ones((n, 1), jnp.float32)
    iota = jax.lax.broadcasted_iota(jnp.int32, (n, K_PROTO), 1)
    for it in range(N_REFINE):
        step = jnp.float32(float(LR_REFINE) / (1.0 + it * 0.5))
        sim = jax.lax.dot_general(
            feats, p, (((1,), (1,)), ((), ())),
            preferred_element_type=jnp.float32)
        mx = jnp.max(sim, axis=1, keepdims=True)
        assign = jnp.min(jnp.where(sim >= mx, iota, K_PROTO), axis=1,
                         keepdims=True)
        onehot = (iota == assign).astype(jnp.float32)
        sums = jax.lax.dot_general(
            onehot, feats, (((0,), (0,)), ((), ())),
            preferred_element_type=jnp.float32)
        counts = jax.lax.dot_general(
            onehot, ones, (((0,), (0,)), ((), ())),
            preferred_element_type=jnp.float32)
        mean = sums / jnp.maximum(counts, 1.0)
        cand = _safe_norm((1.0 - step) * p + step * mean)
        p = jnp.where(counts > 0.0, cand, p)
    return p


def _lse_rows(x):
    mx = jnp.max(x, axis=1, keepdims=True)
    return mx + jnp.log(jnp.sum(jnp.exp(x - mx), axis=1, keepdims=True))


def _refine_body(fg_ref, bg_ref, pos_ref, neg_ref,
                 loss_ref, rfg_ref, rbg_ref):
    fg0 = fg_ref[...]
    bg0 = bg_ref[...]
    pos = pos_ref[...]
    neg = neg_ref[...]

    p_fg = _refine_unrolled(fg0, pos)
    p_bg = _refine_unrolled(bg0, neg)

    pos_n = _safe_norm(pos)
    neg_n = _safe_norm(neg)
    sim_a = jax.lax.dot_general(
        pos_n, p_fg, (((1,), (1,)), ((), ())),
        preferred_element_type=jnp.float32)
    sim_b = jax.lax.dot_general(
        neg_n, p_fg, (((1,), (1,)), ((), ())),
        preferred_element_type=jnp.float32)
    sim_pos = jnp.mean(jnp.max(sim_a, axis=1))
    sim_neg = jnp.mean(jnp.max(sim_b, axis=1))
    triplet = jnp.maximum(MARGIN + sim_neg - sim_pos, 0.0)

    sim_c = jax.lax.dot_general(
        pos_n, p_bg, (((1,), (1,)), ((), ())),
        preferred_element_type=jnp.float32)
    num = _lse_rows(sim_a / TAU)
    lse_n = _lse_rows(sim_c / TAU)
    m2 = jnp.maximum(num, lse_n)
    den = m2 + jnp.log(jnp.exp(num - m2) + jnp.exp(lse_n - m2))
    infonce = -jnp.mean(num - den)

    loss_ref[...] = jnp.broadcast_to(triplet + 0.25 * infonce, (1, 1))
    rfg_ref[...] = _safe_norm((1.0 - M_COEF) * fg0 + M_COEF * p_fg)
    rbg_ref[...] = _safe_norm((1.0 - M_COEF) * bg0 + M_COEF * p_bg)


def _refine_and_loss(fg_proto, bg_proto, pos_feats, neg_feats):
    full = lambda shp: pl.BlockSpec(shp, lambda: tuple(0 for _ in shp))
    return pl.pallas_call(
        _refine_body,
        in_specs=[
            full((K_PROTO, D_FEAT)),
            full((K_PROTO, D_FEAT)),
            full((N_SAMPLES, D_FEAT)),
            full((N_SAMPLES, D_FEAT)),
        ],
        out_specs=[
            full((1, 1)),
            full((K_PROTO, D_FEAT)),
            full((K_PROTO, D_FEAT)),
        ],
        out_shape=[
            jax.ShapeDtypeStruct((1, 1), jnp.float32),
            jax.ShapeDtypeStruct((K_PROTO, D_FEAT), jnp.float32),
            jax.ShapeDtypeStruct((K_PROTO, D_FEAT), jnp.float32),
        ],
    )(fg_proto, bg_proto, pos_feats, neg_feats)


def kernel(fg_proto, bg_proto, F_map, M_map):
    n, c, h, w = F_map.shape
    ft = jnp.transpose(F_map, (0, 2, 3, 1)).reshape(-1, c)
    m_col = jnp.clip(M_map.astype(jnp.float32), 0.0, 1.0).reshape(-1, 1)

    protos = jnp.concatenate([fg_proto, bg_proto], axis=0)
    f_norm, scores = _scores_and_norm(ft, protos, m_col)

    k = min(N_SAMPLES, ft.shape[0])
    _, pos_idx = jax.lax.top_k(scores[:, 0], k)
    _, neg_idx = jax.lax.top_k(scores[:, 1], k)
    pos_feats = f_norm[pos_idx]
    neg_feats = f_norm[neg_idx]

    loss, rfg, rbg = _refine_and_loss(fg_proto, bg_proto, pos_feats, neg_feats)
    return (loss.reshape(()), rfg, rbg)
